# Initial kernel scaffold; baseline (speedup 1.0000x reference)
import jax
import jax.numpy as jnp
from jax.experimental import pallas as pl


def kernel(embeddings, batch_indices):
    raise NotImplementedError("write your pallas kernel here")


# speedup vs baseline: 1.0003x; 1.0003x over previous
import jax
import jax.numpy as jnp
from jax.experimental import pallas as pl

_K = 1024
_ITERS = 10


def _norm(v, axis=-1, eps=1e-12):
    return v / (jnp.linalg.norm(v, axis=axis, keepdims=True) + eps)


def kernel(embeddings, batch_indices):
    key = jax.random.key(42)
    embs = embeddings.reshape(-1, embeddings.shape[-1])
    n = embs.shape[0]
    key, sub = jax.random.split(key)
    init_labels = jax.random.randint(sub, (n,), 0, _K)
    x = _norm(embs)
    labels = init_labels
    for _ in range(_ITERS):
        centroids = jax.ops.segment_sum(x, labels, num_segments=_K)
        centroids = _norm(centroids)
        sims = x @ centroids.T
        labels = jnp.argmax(sims, axis=1)
    return (labels, batch_indices.astype(labels.dtype))


# speedup vs baseline: 1.0180x; 1.0177x over previous
import functools

import jax
import jax.numpy as jnp
from jax.experimental import pallas as pl
from jax.experimental.pallas import tpu as pltpu

_K = 1024
_ITERS = 10
_N = 16384
_D = 256
_ROWS = 1024
_NBLK = _N // _ROWS


def _assign_body(craw_ref, x_ref, out_ref, cn_ref):
    @pl.when(pl.program_id(0) == 0)
    def _():
        c = craw_ref[...]
        norm = jnp.sqrt(jnp.sum(c * c, axis=1, keepdims=True))
        cn_ref[...] = c / (norm + 1e-12)

    sims = jax.lax.dot_general(
        x_ref[...], cn_ref[...],
        dimension_numbers=(((1,), (1,)), ((), ())),
        preferred_element_type=jnp.float32)
    out_ref[...] = jnp.argmax(sims, axis=1).astype(jnp.int32).reshape(1, 1, _ROWS)


@jax.jit
def _tc_assign(xn, craw):
    out = pl.pallas_call(
        _assign_body,
        grid=(_NBLK,),
        in_specs=[
            pl.BlockSpec((_K, _D), lambda i: (0, 0)),
            pl.BlockSpec((_ROWS, _D), lambda i: (i, 0)),
        ],
        out_specs=pl.BlockSpec((1, 1, _ROWS), lambda i: (i, 0, 0)),
        out_shape=jax.ShapeDtypeStruct((_NBLK, 1, _ROWS), jnp.int32),
        scratch_shapes=[pltpu.VMEM((_K, _D), jnp.float32)],
    )(craw, xn)
    return out.reshape(_N)


def _norm(v, axis=-1, eps=1e-12):
    return v / (jnp.linalg.norm(v, axis=axis, keepdims=True) + eps)


def kernel(embeddings, batch_indices):
    key = jax.random.key(42)
    embs = embeddings.reshape(-1, embeddings.shape[-1])
    n = embs.shape[0]
    key, sub = jax.random.split(key)
    init_labels = jax.random.randint(sub, (n,), 0, _K)
    x = _norm(embs)
    labels = init_labels
    for _ in range(_ITERS):
        craw = jax.ops.segment_sum(x, labels, num_segments=_K)
        labels = _tc_assign(x, craw)
    return (labels, batch_indices.astype(labels.dtype))
